# baseline (speedup 1.0000x reference)
import functools

import jax
import jax.numpy as jnp
from jax import lax
from jax.experimental import pallas as pl
from jax.experimental.pallas import tpu as pltpu
from jax.experimental.pallas import tpu_sc as plsc

TOPK = 20
TEMP = 0.1
CHUNK = 128
NEG = -1e30
_SC_NC = 2
_SC_NS = 16
_SC_L = 16


def _phase_a_body(k_real, t_tile, n_k, q_ref, b_ref, sim_ref, m_ref):
    q = q_ref[...]
    nrm = jnp.sqrt(jnp.sum(q * q, axis=1, keepdims=True))
    nrm = jnp.maximum(nrm, 1e-12)
    qn = q / nrm
    b = b_ref[...]
    sim = jax.lax.dot_general(
        qn, b, (((1,), (1,)), ((), ())), preferred_element_type=jnp.float32
    )
    ki = pl.program_id(0)
    qb = sim.shape[0]

    def commit(s):
        sim_ref[...] = s
        m_ref[0] = jnp.max(s.reshape(qb, t_tile // CHUNK, CHUNK), axis=-1)

    @pl.when(ki < n_k - 1)
    def _():
        commit(sim)

    @pl.when(ki == n_k - 1)
    def _():
        cols = ki * t_tile + jax.lax.broadcasted_iota(jnp.int32, sim.shape, 1)
        commit(jnp.where(cols < k_real, sim, NEG))


def _phase_a(qfeats, bank, k_real, qb, t_tile):
    qn_rows, d = qfeats.shape
    kp = bank.shape[0]
    n_k = kp // t_tile
    n_q = qn_rows // qb
    sim, m = pl.pallas_call(
        functools.partial(_phase_a_body, k_real, t_tile, n_k),
        grid=(n_k, n_q),
        in_specs=[
            pl.BlockSpec((qb, d), lambda ki, qi: (qi, 0)),
            pl.BlockSpec((t_tile, d), lambda ki, qi: (ki, 0)),
        ],
        out_specs=[
            pl.BlockSpec((qb, t_tile), lambda ki, qi: (qi, ki)),
            pl.BlockSpec((1, qb, t_tile // CHUNK), lambda ki, qi: (ki, qi, 0)),
        ],
        out_shape=[
            jax.ShapeDtypeStruct((qn_rows, kp), jnp.float32),
            jax.ShapeDtypeStruct((n_k, qn_rows, t_tile // CHUNK), jnp.float32),
        ],
        compiler_params=pltpu.CompilerParams(
            dimension_semantics=("arbitrary", "arbitrary"),
        ),
    )(qfeats, bank)
    return sim, m


def _phase_b_body(qb, nchunk, m_ref, ids_ref):
    cur = m_ref[...]
    iota = jax.lax.broadcasted_iota(jnp.int32, (qb, nchunk), 1)
    ids_cols = []
    for _ in range(TOPK):
        mx = jnp.max(cur, axis=1, keepdims=True)
        idxc = jnp.where(cur >= mx, iota, jnp.int32(1 << 30))
        amn = jnp.min(idxc, axis=1, keepdims=True)
        ids_cols.append(amn)
        cur = jnp.where(iota == amn, NEG, cur)
    qi = pl.program_id(0)
    qrow = qi * qb + jax.lax.broadcasted_iota(jnp.int32, (qb, TOPK), 0)
    ids_ref[...] = jnp.concatenate(ids_cols, axis=1) + qrow * nchunk


def _phase_b(m, qb):
    q_rows, nchunk = m.shape
    n_q = q_rows // qb
    return pl.pallas_call(
        functools.partial(_phase_b_body, qb, nchunk),
        grid=(n_q,),
        in_specs=[pl.BlockSpec((qb, nchunk), lambda qi: (qi, 0))],
        out_specs=pl.BlockSpec((qb, TOPK), lambda qi: (qi, 0)),
        out_shape=jax.ShapeDtypeStruct((q_rows, TOPK), jnp.int32),
    )(m)


def _sc_gather_cand(simf, idsf, q_rows):
    NW = _SC_NC * _SC_NS
    QPW = q_rows // NW
    QB4 = 4
    NB = QPW // QB4
    NR = QB4 * TOPK

    mesh = plsc.VectorSubcoreMesh(core_axis_name="c", subcore_axis_name="s")

    @functools.partial(
        pl.kernel,
        mesh=mesh,
        out_type=jax.ShapeDtypeStruct((q_rows * TOPK, CHUNK), jnp.float32),
        scratch_types=[
            pltpu.VMEM((QPW * TOPK,), jnp.int32),
            pltpu.VMEM((NR, CHUNK), jnp.float32),
            pltpu.SemaphoreType.DMA,
        ],
    )
    def body(simf_h, ids_h, cand_h, ids_v, chunks, sem):
        wid = lax.axis_index("s") * _SC_NC + lax.axis_index("c")
        qbase = wid * QPW
        pltpu.sync_copy(ids_h.at[pl.ds(qbase * TOPK, QPW * TOPK)], ids_v)

        def batch_body(b, carry):
            pltpu.async_copy(
                simf_h.at[ids_v.at[pl.ds(b * NR, NR)]], chunks, sem).wait()
            pltpu.sync_copy(
                chunks, cand_h.at[pl.ds(qbase * TOPK + b * NR, NR)])
            return carry

        lax.fori_loop(0, NB, batch_body, 0)

    return body(simf, idsf)


def _phase_c2_body(qb, ncand, nchunk, cand_ref, ids_ref, w_ref, bidx_ref):
    cur = cand_ref[...]
    idsm = ids_ref[...]
    iota_c = jax.lax.broadcasted_iota(jnp.int32, (qb, ncand), 1)
    vals_cols, j_cols = [], []
    for _ in range(TOPK):
        mx = jnp.max(cur, axis=1, keepdims=True)
        idxc = jnp.where(cur >= mx, iota_c, jnp.int32(1 << 30))
        amn = jnp.min(idxc, axis=1, keepdims=True)
        vals_cols.append(mx)
        j_cols.append(amn)
        cur = jnp.where(iota_c == amn, NEG, cur)
    tv = jnp.concatenate(vals_cols, axis=1)
    tj = jnp.concatenate(j_cols, axis=1)
    slot = tj >> 7
    cid = jnp.zeros((qb, TOPK), jnp.int32)
    for k in range(TOPK):
        cid = cid + jnp.where(slot == k, idsm[:, k:k + 1], 0)
    qi = pl.program_id(0)
    qrow = qi * qb + jax.lax.broadcasted_iota(jnp.int32, (qb, TOPK), 0)
    bidx_ref[...] = (cid - qrow * nchunk) * CHUNK + (tj & (CHUNK - 1))
    e = jnp.exp((tv - tv[:, 0:1]) * (1.0 / TEMP))
    w_ref[...] = e / jnp.sum(e, axis=1, keepdims=True)


def _phase_c2(cand, ids, qb, nchunk):
    q_rows, ncand = cand.shape
    n_q = q_rows // qb
    w, bidx = pl.pallas_call(
        functools.partial(_phase_c2_body, qb, ncand, nchunk),
        grid=(n_q,),
        in_specs=[
            pl.BlockSpec((qb, ncand), lambda qi: (qi, 0)),
            pl.BlockSpec((qb, TOPK), lambda qi: (qi, 0)),
        ],
        out_specs=[
            pl.BlockSpec((qb, TOPK), lambda qi: (qi, 0)),
            pl.BlockSpec((qb, TOPK), lambda qi: (qi, 0)),
        ],
        out_shape=[
            jax.ShapeDtypeStruct((q_rows, TOPK), jnp.float32),
            jax.ShapeDtypeStruct((q_rows, TOPK), jnp.int32),
        ],
    )(cand, ids)
    return w, bidx


def _sc_combine(wf, bidxf, coords128, q_rows):
    L = _SC_L
    NW = _SC_NC * _SC_NS
    QPW = q_rows // NW
    QB4 = 4
    NB = QPW // QB4
    NR = QB4 * TOPK

    mesh = plsc.VectorSubcoreMesh(core_axis_name="c", subcore_axis_name="s")

    @functools.partial(
        pl.kernel,
        mesh=mesh,
        out_type=jax.ShapeDtypeStruct((q_rows * L,), jnp.float32),
        scratch_types=[
            pltpu.VMEM((QPW * TOPK,), jnp.float32),
            pltpu.VMEM((QPW * TOPK,), jnp.int32),
            pltpu.VMEM((NR, CHUNK), jnp.float32),
            pltpu.VMEM((QPW * L,), jnp.float32),
            pltpu.SemaphoreType.DMA,
        ],
    )
    def body(w_h, bidx_h, coords_h, out_h, w_v, b_v, cbuf, obuf, sem):
        iota16 = lax.iota(jnp.int32, L)
        gdn = lax.GatherDimensionNumbers(
            offset_dims=(), collapsed_slice_dims=(0,), start_index_map=(0,))

        def bcast_lane(vec, j):
            idx = jnp.full((L,), j, jnp.int32)
            return lax.gather(vec, idx[:, None], gdn, (1,),
                              mode=lax.GatherScatterMode.PROMISE_IN_BOUNDS)

        wid = lax.axis_index("s") * _SC_NC + lax.axis_index("c")
        qbase = wid * QPW
        pltpu.sync_copy(w_h.at[pl.ds(qbase * TOPK, QPW * TOPK)], w_v)
        pltpu.sync_copy(bidx_h.at[pl.ds(qbase * TOPK, QPW * TOPK)], b_v)

        def batch_body(b, carry):
            pltpu.async_copy(
                coords_h.at[b_v.at[pl.ds(b * NR, NR)]], cbuf, sem).wait()
            for qq in range(QB4):
                qloc = b * QB4 + qq
                wa = w_v[pl.ds(qloc * TOPK, L)]
                wb = w_v[pl.ds(qloc * TOPK + 4, L)]
                acc = jnp.zeros((L,), jnp.float32)
                for j in range(TOPK):
                    wj = bcast_lane(wa, j) if j < 16 else bcast_lane(wb, j - 4)
                    acc = acc + wj * cbuf[qq * TOPK + j, pl.ds(0, L)]
                obuf[pl.ds(qloc * L, L)] = acc
            return carry

        lax.fori_loop(0, NB, batch_body, 0)
        pltpu.sync_copy(obuf, out_h.at[pl.ds(qbase * L, QPW * L)])

    return body(wf, bidxf, coords128)


def kernel(test_feats, bank_feats, bank_coords, lat_mean, lat_std, lon_mean, lon_std):
    q_rows, d = test_feats.shape
    k_real = bank_feats.shape[0]

    t_tile = 4096 if k_real > 4096 else 512
    qb = 256 if q_rows >= 256 else q_rows
    kp = ((k_real + t_tile - 1) // t_tile) * t_tile
    bank = jnp.concatenate(
        [bank_feats, jnp.zeros((kp - k_real, d), jnp.float32)], axis=0
    )

    sim, m3 = _phase_a(test_feats, bank, k_real, qb, t_tile)
    m = m3.transpose(1, 0, 2).reshape(q_rows, kp // CHUNK)

    nchunk = kp // CHUNK
    ids = _phase_b(m, qb)

    simf = sim.reshape(q_rows * nchunk, CHUNK)
    cand = _sc_gather_cand(simf, ids.reshape(-1), q_rows)

    w, bidx = _phase_c2(cand.reshape(q_rows, TOPK * CHUNK), ids, 128, nchunk)

    coords128 = jnp.pad(bank_coords, ((0, 0), (0, CHUNK - 2)))
    outv = _sc_combine(w.reshape(-1), bidx.reshape(-1), coords128, q_rows)
    pred = outv.reshape(q_rows, _SC_L)
    lat = pred[:, 0] * lat_std + lat_mean
    lon = pred[:, 1] * lon_std + lon_mean
    return jnp.stack([lat, lon], axis=1)


# speedup vs baseline: 1.8308x; 1.8308x over previous
import functools

import jax
import jax.numpy as jnp
from jax import lax
from jax.experimental import pallas as pl
from jax.experimental.pallas import tpu as pltpu
from jax.experimental.pallas import tpu_sc as plsc

TOPK = 20
TEMP = 0.1
CHUNK = 128
NEG = -1e30
_SC_NC = 2
_SC_NS = 16
_SC_L = 16


def _phase_a_body(k_real, t_tile, n_k, q_ref, b_ref, sim_ref, m_ref):
    q = q_ref[...]
    nrm = jnp.sqrt(jnp.sum(q * q, axis=1, keepdims=True))
    nrm = jnp.maximum(nrm, 1e-12)
    qn = q / nrm
    b = b_ref[...]
    sim = jax.lax.dot_general(
        qn, b, (((1,), (1,)), ((), ())), preferred_element_type=jnp.float32
    )
    ki = pl.program_id(0)
    qb = sim.shape[0]
    cols = ki * t_tile + jax.lax.broadcasted_iota(jnp.int32, sim.shape, 1)
    sim = jnp.where(cols < k_real, sim, NEG)
    sim3 = sim.reshape(qb, t_tile // CHUNK, CHUNK)
    sim_ref[...] = sim3
    m_ref[0] = jnp.max(sim3, axis=-1)


def _phase_a(qfeats, bank, k_real, qb, t_tile):
    qn_rows, d = qfeats.shape
    kp = bank.shape[0]
    n_k = kp // t_tile
    n_q = qn_rows // qb
    sim, m = pl.pallas_call(
        functools.partial(_phase_a_body, k_real, t_tile, n_k),
        grid=(n_k, n_q),
        in_specs=[
            pl.BlockSpec((qb, d), lambda ki, qi: (qi, 0)),
            pl.BlockSpec((t_tile, d), lambda ki, qi: (ki, 0)),
        ],
        out_specs=[
            pl.BlockSpec((qb, t_tile // CHUNK, CHUNK),
                         lambda ki, qi: (qi, ki, 0)),
            pl.BlockSpec((1, qb, t_tile // CHUNK), lambda ki, qi: (ki, qi, 0)),
        ],
        out_shape=[
            jax.ShapeDtypeStruct((qn_rows, kp // CHUNK, CHUNK), jnp.float32),
            jax.ShapeDtypeStruct((n_k, qn_rows, t_tile // CHUNK), jnp.float32),
        ],
        compiler_params=pltpu.CompilerParams(
            dimension_semantics=("arbitrary", "arbitrary"),
        ),
    )(qfeats, bank)
    return sim, m


def _phase_b_body(qb, nchunk, m_ref, ids_ref):
    cur = m_ref[...]
    iota = jax.lax.broadcasted_iota(jnp.int32, (qb, nchunk), 1)
    ids_cols = []
    for _ in range(TOPK):
        mx = jnp.max(cur, axis=1, keepdims=True)
        idxc = jnp.where(cur >= mx, iota, jnp.int32(1 << 30))
        amn = jnp.min(idxc, axis=1, keepdims=True)
        ids_cols.append(amn)
        cur = jnp.where(iota == amn, NEG, cur)
    qi = pl.program_id(0)
    qrow = qi * qb + jax.lax.broadcasted_iota(jnp.int32, (qb, TOPK), 0)
    ids_ref[...] = jnp.concatenate(ids_cols, axis=1) + qrow * nchunk


def _phase_b(m, qb):
    q_rows, nchunk = m.shape
    n_q = q_rows // qb
    return pl.pallas_call(
        functools.partial(_phase_b_body, qb, nchunk),
        grid=(n_q,),
        in_specs=[pl.BlockSpec((qb, nchunk), lambda qi: (qi, 0))],
        out_specs=pl.BlockSpec((qb, TOPK), lambda qi: (qi, 0)),
        out_shape=jax.ShapeDtypeStruct((q_rows, TOPK), jnp.int32),
    )(m)


def _sc_gather_cand(simf, idsf, q_rows):
    NW = _SC_NC * _SC_NS
    QPW = q_rows // NW
    QB4 = 4
    NB = QPW // QB4
    NR = QB4 * TOPK

    mesh = plsc.VectorSubcoreMesh(core_axis_name="c", subcore_axis_name="s")

    @functools.partial(
        pl.kernel,
        mesh=mesh,
        out_type=jax.ShapeDtypeStruct((q_rows * TOPK, CHUNK), jnp.float32),
        scratch_types=[
            pltpu.VMEM((QPW * TOPK,), jnp.int32),
            pltpu.VMEM((NR, CHUNK), jnp.float32),
            pltpu.SemaphoreType.DMA,
        ],
    )
    def body(simf_h, ids_h, cand_h, ids_v, chunks, sem):
        wid = lax.axis_index("s") * _SC_NC + lax.axis_index("c")
        qbase = wid * QPW
        pltpu.sync_copy(ids_h.at[pl.ds(qbase * TOPK, QPW * TOPK)], ids_v)

        def batch_body(b, carry):
            pltpu.async_copy(
                simf_h.at[ids_v.at[pl.ds(b * NR, NR)]], chunks, sem).wait()
            pltpu.sync_copy(
                chunks, cand_h.at[pl.ds(qbase * TOPK + b * NR, NR)])
            return carry

        lax.fori_loop(0, NB, batch_body, 0)

    return body(simf, idsf)


def _phase_c2_body(qb, ncand, nchunk, cand_ref, ids_ref, w_ref, bidx_ref):
    cur = cand_ref[...]
    idsm = ids_ref[...]
    iota_c = jax.lax.broadcasted_iota(jnp.int32, (qb, ncand), 1)
    vals_cols, j_cols = [], []
    for _ in range(TOPK):
        mx = jnp.max(cur, axis=1, keepdims=True)
        idxc = jnp.where(cur >= mx, iota_c, jnp.int32(1 << 30))
        amn = jnp.min(idxc, axis=1, keepdims=True)
        vals_cols.append(mx)
        j_cols.append(amn)
        cur = jnp.where(iota_c == amn, NEG, cur)
    tv = jnp.concatenate(vals_cols, axis=1)
    tj = jnp.concatenate(j_cols, axis=1)
    slot = tj >> 7
    cid = jnp.zeros((qb, TOPK), jnp.int32)
    for k in range(TOPK):
        cid = cid + jnp.where(slot == k, idsm[:, k:k + 1], 0)
    qi = pl.program_id(0)
    qrow = qi * qb + jax.lax.broadcasted_iota(jnp.int32, (qb, TOPK), 0)
    bidx_ref[...] = (cid - qrow * nchunk) * CHUNK + (tj & (CHUNK - 1))
    e = jnp.exp((tv - tv[:, 0:1]) * (1.0 / TEMP))
    w_ref[...] = e / jnp.sum(e, axis=1, keepdims=True)


def _phase_c2(cand, ids, qb, nchunk):
    q_rows, ncand = cand.shape
    n_q = q_rows // qb
    w, bidx = pl.pallas_call(
        functools.partial(_phase_c2_body, qb, ncand, nchunk),
        grid=(n_q,),
        in_specs=[
            pl.BlockSpec((qb, ncand), lambda qi: (qi, 0)),
            pl.BlockSpec((qb, TOPK), lambda qi: (qi, 0)),
        ],
        out_specs=[
            pl.BlockSpec((qb, TOPK), lambda qi: (qi, 0)),
            pl.BlockSpec((qb, TOPK), lambda qi: (qi, 0)),
        ],
        out_shape=[
            jax.ShapeDtypeStruct((q_rows, TOPK), jnp.float32),
            jax.ShapeDtypeStruct((q_rows, TOPK), jnp.int32),
        ],
    )(cand, ids)
    return w, bidx


def _sc_combine(wf, bidxf, coords128, q_rows):
    L = _SC_L
    NW = _SC_NC * _SC_NS
    QPW = q_rows // NW
    QB4 = 4
    NB = QPW // QB4
    NR = QB4 * TOPK

    mesh = plsc.VectorSubcoreMesh(core_axis_name="c", subcore_axis_name="s")

    @functools.partial(
        pl.kernel,
        mesh=mesh,
        out_type=jax.ShapeDtypeStruct((q_rows * L,), jnp.float32),
        scratch_types=[
            pltpu.VMEM((QPW * TOPK,), jnp.float32),
            pltpu.VMEM((QPW * TOPK,), jnp.int32),
            pltpu.VMEM((NR, CHUNK), jnp.float32),
            pltpu.VMEM((QPW * L,), jnp.float32),
            pltpu.SemaphoreType.DMA,
        ],
    )
    def body(w_h, bidx_h, coords_h, out_h, w_v, b_v, cbuf, obuf, sem):
        iota16 = lax.iota(jnp.int32, L)
        gdn = lax.GatherDimensionNumbers(
            offset_dims=(), collapsed_slice_dims=(0,), start_index_map=(0,))

        def bcast_lane(vec, j):
            idx = jnp.full((L,), j, jnp.int32)
            return lax.gather(vec, idx[:, None], gdn, (1,),
                              mode=lax.GatherScatterMode.PROMISE_IN_BOUNDS)

        wid = lax.axis_index("s") * _SC_NC + lax.axis_index("c")
        qbase = wid * QPW
        pltpu.sync_copy(w_h.at[pl.ds(qbase * TOPK, QPW * TOPK)], w_v)
        pltpu.sync_copy(bidx_h.at[pl.ds(qbase * TOPK, QPW * TOPK)], b_v)

        def batch_body(b, carry):
            pltpu.async_copy(
                coords_h.at[b_v.at[pl.ds(b * NR, NR)]], cbuf, sem).wait()
            for qq in range(QB4):
                qloc = b * QB4 + qq
                wa = w_v[pl.ds(qloc * TOPK, L)]
                wb = w_v[pl.ds(qloc * TOPK + 4, L)]
                acc = jnp.zeros((L,), jnp.float32)
                for j in range(TOPK):
                    wj = bcast_lane(wa, j) if j < 16 else bcast_lane(wb, j - 4)
                    acc = acc + wj * cbuf[qq * TOPK + j, pl.ds(0, L)]
                obuf[pl.ds(qloc * L, L)] = acc
            return carry

        lax.fori_loop(0, NB, batch_body, 0)
        pltpu.sync_copy(obuf, out_h.at[pl.ds(qbase * L, QPW * L)])

    return body(wf, bidxf, coords128)


def kernel(test_feats, bank_feats, bank_coords, lat_mean, lat_std, lon_mean, lon_std):
    q_rows, d = test_feats.shape
    k_real = bank_feats.shape[0]

    t_tile = 4096 if k_real > 4096 else 512
    qb = 256 if q_rows >= 256 else q_rows
    kp = ((k_real + t_tile - 1) // t_tile) * t_tile
    bank = jnp.concatenate(
        [bank_feats, jnp.zeros((kp - k_real, d), jnp.float32)], axis=0
    )

    sim, m3 = _phase_a(test_feats, bank, k_real, qb, t_tile)
    m = m3.transpose(1, 0, 2).reshape(q_rows, kp // CHUNK)

    nchunk = kp // CHUNK
    ids = _phase_b(m, qb)

    simf = sim.reshape(q_rows * nchunk, CHUNK)
    cand = _sc_gather_cand(simf, ids.reshape(-1), q_rows)

    w, bidx = _phase_c2(cand.reshape(q_rows, TOPK * CHUNK), ids, 128, nchunk)

    coords128 = jnp.pad(bank_coords, ((0, 0), (0, CHUNK - 2)))
    outv = _sc_combine(w.reshape(-1), bidx.reshape(-1), coords128, q_rows)
    pred = outv.reshape(q_rows, _SC_L)
    lat = pred[:, 0] * lat_std + lat_mean
    lon = pred[:, 1] * lon_std + lon_mean
    return jnp.stack([lat, lon], axis=1)
